# Initial kernel scaffold; baseline (speedup 1.0000x reference)
import functools

import jax
import jax.numpy as jnp
from jax import lax
from jax.experimental import pallas as pl
from jax.experimental.pallas import tpu as pltpu
from jax.experimental.pallas import tpu_sc as plsc

N = 10000
E = 160000
D = 256
H = 128
N_PAD = 10240
RPT = N_PAD // 16
CHUNK = 128
E_PER_TILE = 10240
CHUNKS = E_PER_TILE // CHUNK
E_PAD = E_PER_TILE * 16
BN = 512


def _make_sc_aggregate(compute_deg: bool):
    mesh = plsc.VectorSubcoreMesh(core_axis_name="c", subcore_axis_name="s")
    out_type = [jax.ShapeDtypeStruct((2, N_PAD, H), jnp.float32)]
    if compute_deg:
        out_type.append(jax.ShapeDtypeStruct((N_PAD, 16), jnp.float32))
    scratch = [
        pltpu.VMEM((CHUNKS, CHUNK), jnp.int32),
        pltpu.VMEM((CHUNKS, CHUNK), jnp.int32),
        pltpu.VMEM((CHUNK, H), jnp.float32),
        pltpu.VMEM((16, H), jnp.float32),
        pltpu.VMEM((16, 16), jnp.float32),
        pltpu.VMEM((CHUNK, 16), jnp.float32),
        pltpu.VMEM_SHARED((N_PAD, H), jnp.float32),
        pltpu.VMEM_SHARED((N_PAD, 16), jnp.float32),
        pltpu.SemaphoreType.DMA,
    ]

    def body(x2n, src_idx, dst_idx, *rest):
        if compute_deg:
            agg_out, deg_out = rest[0], rest[1]
            rest = rest[2:]
        else:
            agg_out, deg_out = rest[0], None
            rest = rest[1:]
        src_v, dst_v, rows_v, zb, zd, ones_v, agg_sh, deg_sh, sem = rest

        c = lax.axis_index("c")
        s = lax.axis_index("s")
        base = s * RPT

        zeros16 = jnp.zeros((16,), jnp.float32)
        ones16 = jnp.ones((16,), jnp.float32)
        for r in range(16):
            for k in range(H // 16):
                zb[r, pl.ds(k * 16, 16)] = zeros16
            zd[r, pl.ds(0, 16)] = zeros16
        if compute_deg:
            for r in range(CHUNK):
                ones_v[r, pl.ds(0, 16)] = ones16

        def zero_agg(i, carry):
            pltpu.sync_copy(zb, agg_sh.at[pl.ds(base + i * 16, 16)])
            if compute_deg:
                pltpu.sync_copy(zd, deg_sh.at[pl.ds(base + i * 16, 16)])
            return carry

        lax.fori_loop(0, RPT // 16, zero_agg, 0)

        pltpu.sync_copy(src_idx.at[c, s], src_v)
        pltpu.sync_copy(dst_idx.at[s], dst_v)

        plsc.subcore_barrier()

        def chunk(j, carry):
            pltpu.async_copy(x2n.at[src_v.at[j]], rows_v, sem).wait()
            pltpu.sync_copy(rows_v, agg_sh.at[dst_v.at[j]], add=True)
            if compute_deg:
                pltpu.sync_copy(ones_v, deg_sh.at[dst_v.at[j]], add=True)
            return carry

        lax.fori_loop(0, CHUNKS, chunk, 0)

        plsc.subcore_barrier()

        pltpu.sync_copy(agg_sh.at[pl.ds(base, RPT)],
                        agg_out.at[c, pl.ds(base, RPT)])
        if compute_deg:
            @pl.when(c == 0)
            def _():
                pltpu.sync_copy(deg_sh.at[pl.ds(base, RPT)],
                                deg_out.at[pl.ds(base, RPT)])

    return pl.kernel(body, out_type=out_type, mesh=mesh,
                     scratch_types=scratch)


def _dense_body(paired_out, agg_ref, deg_ref, xr_ref, wl_ref, wr_ref, b_ref,
                out_ref):
    aggf = jnp.concatenate([agg_ref[0], agg_ref[1]], axis=1)
    xf = jnp.concatenate([xr_ref[0], xr_ref[1]], axis=1)
    inv = 1.0 / jnp.maximum(deg_ref[:, 0:1], 1.0)
    h = jnp.dot(aggf * inv, wl_ref[...], preferred_element_type=jnp.float32)
    h = h + b_ref[...] + jnp.dot(xf, wr_ref[...],
                                 preferred_element_type=jnp.float32)
    h = jnp.maximum(h, 0.0)
    if paired_out:
        out_ref[0] = h[:, :H]
        out_ref[1] = h[:, H:]
    else:
        out_ref[...] = h


def _make_dense(paired_out: bool):
    grid = (N_PAD // BN,)
    pair_spec = pl.BlockSpec((2, BN, H), lambda i: (0, i, 0))
    in_specs = [
        pair_spec,
        pl.BlockSpec((BN, 16), lambda i: (i, 0)),
        pair_spec,
        pl.BlockSpec((D, D), lambda i: (0, 0)),
        pl.BlockSpec((D, D), lambda i: (0, 0)),
        pl.BlockSpec((1, D), lambda i: (0, 0)),
    ]
    if paired_out:
        out_shape = jax.ShapeDtypeStruct((2, N_PAD, H), jnp.float32)
        out_specs = pair_spec
    else:
        out_shape = jax.ShapeDtypeStruct((N_PAD, D), jnp.float32)
        out_specs = pl.BlockSpec((BN, D), lambda i: (i, 0))
    return pl.pallas_call(
        functools.partial(_dense_body, paired_out),
        grid=grid, in_specs=in_specs, out_specs=out_specs,
        out_shape=out_shape)


_sc_agg_deg = _make_sc_aggregate(True)
_sc_agg = _make_sc_aggregate(False)
_dense_paired = _make_dense(True)
_dense_flat = _make_dense(False)


def kernel(x, edge_index, W1_l, b1, W1_r, W2_l, b2, W2_r):
    src = edge_index[0].astype(jnp.int32)
    dst = edge_index[1].astype(jnp.int32)
    src_p = jnp.concatenate([src, jnp.zeros((E_PAD - E,), jnp.int32)])
    dst_p = jnp.concatenate([dst, jnp.full((E_PAD - E,), N, jnp.int32)])
    src2 = jnp.stack([src_p, src_p + N_PAD]).reshape(2, 16, CHUNKS, CHUNK)
    dst3 = dst_p.reshape(16, CHUNKS, CHUNK)

    xt = x.reshape(N, 2, H).transpose(1, 0, 2)
    xt = jnp.pad(xt, ((0, 0), (0, N_PAD - N), (0, 0)))
    x2n = xt.reshape(2 * N_PAD, H)

    agg1, deg = _sc_agg_deg(x2n, src2, dst3)
    h2n = _dense_paired(agg1, deg, x2n.reshape(2, N_PAD, H),
                        W1_l.T, W1_r.T, b1.reshape(1, D))
    agg2 = _sc_agg(h2n.reshape(2 * N_PAD, H), src2, dst3)
    out = _dense_flat(agg2, deg, h2n, W2_l.T, W2_r.T, b2.reshape(1, D))
    return out[:N]


# speedup vs baseline: 3.2050x; 3.2050x over previous
import functools

import jax
import jax.numpy as jnp
from jax import lax
from jax.experimental import pallas as pl
from jax.experimental.pallas import tpu as pltpu
from jax.experimental.pallas import tpu_sc as plsc

N = 10000
E = 160000
D = 256
H = 128
N_PAD = 10240
RPT = N_PAD // 16
CHUNK = 128
GROUPS = 4
GCHUNKS = 20
E_PER_TILE = GROUPS * GCHUNKS * CHUNK
E_PAD = E_PER_TILE * 16
BN = 512


def _make_sc_aggregate(compute_deg: bool):
    mesh = plsc.VectorSubcoreMesh(core_axis_name="c", subcore_axis_name="s",
                                  num_cores=2, num_subcores=16)
    out_type = [jax.ShapeDtypeStruct((2, N_PAD, H), jnp.float32)]
    scratch = [
        pltpu.VMEM((GCHUNKS, CHUNK), jnp.int32),
        pltpu.VMEM((GCHUNKS, CHUNK), jnp.int32),
        pltpu.VMEM((CHUNK, H), jnp.float32),
        pltpu.VMEM_SHARED((N_PAD, H), jnp.float32),
        pltpu.SemaphoreType.DMA,
    ]
    if compute_deg:
        out_type.append(jax.ShapeDtypeStruct((N_PAD,), jnp.float32))
        scratch += [
            pltpu.VMEM((N_PAD,), jnp.float32),
            pltpu.VMEM((RPT,), jnp.float32),
            pltpu.VMEM((RPT,), jnp.float32),
            pltpu.VMEM_SHARED((16, N_PAD), jnp.float32),
        ]

    def body(x2n, src_idx, dst_idx, *rest):
        if compute_deg:
            (agg_out, deg_out, src_v, dst_v, rows_v, agg_sh, sem,
             hist_v, tmp_v, dsum_v, stage_sh) = rest
        else:
            agg_out, src_v, dst_v, rows_v, agg_sh, sem = rest

        c = lax.axis_index("c")
        s = lax.axis_index("s")
        base = s * RPT
        zeros16 = jnp.zeros((16,), jnp.float32)
        for r in range(16):
            for k in range(H // 16):
                rows_v[r, pl.ds(k * 16, 16)] = zeros16

        if compute_deg:
            def zero_hist(g, carry):
                hist_v[pl.ds(g * 16, 16)] = zeros16
                return carry
            lax.fori_loop(0, N_PAD // 16, zero_hist, 0)

        def zero_agg(i, carry):
            pltpu.sync_copy(rows_v.at[pl.ds(0, 16)],
                            agg_sh.at[pl.ds(base + i * 16, 16)])
            return carry
        lax.fori_loop(0, RPT // 16, zero_agg, 0)

        plsc.subcore_barrier()

        def group(g, carry):
            pltpu.sync_copy(src_idx.at[c, s, g], src_v)
            pltpu.sync_copy(dst_idx.at[s, g], dst_v)

            def chunk(j, carry2):
                cp = pltpu.async_copy(x2n.at[src_v.at[j]], rows_v, sem)
                if compute_deg:
                    @pl.when(c == 0)
                    def _():
                        iota16 = lax.iota(jnp.int32, 16)
                        one16 = jnp.ones((16,), jnp.float32)
                        zero16 = jnp.zeros((16,), jnp.float32)
                        for k in range(CHUNK // 16):
                            dvec = dst_v[j, pl.ds(k * 16, 16)]
                            for l in range(16):
                                idx = dvec[l]
                                wbase = lax.bitwise_and(idx, ~15)
                                lane = lax.bitwise_and(idx, 15)
                                oh = jnp.where(iota16 == lane, one16, zero16)
                                w = hist_v[pl.ds(wbase, 16)]
                                hist_v[pl.ds(wbase, 16)] = w + oh
                cp.wait()
                pltpu.sync_copy(rows_v, agg_sh.at[dst_v.at[j]], add=True)
                return carry2
            lax.fori_loop(0, GCHUNKS, chunk, 0)
            return carry
        lax.fori_loop(0, GROUPS, group, 0)

        if compute_deg:
            @pl.when(c == 0)
            def _():
                pltpu.sync_copy(hist_v, stage_sh.at[s])
        plsc.subcore_barrier()

        if compute_deg:
            @pl.when(c == 0)
            def _():
                def zero_dsum(g, carry):
                    dsum_v[pl.ds(g * 16, 16)] = zeros16
                    return carry
                lax.fori_loop(0, RPT // 16, zero_dsum, 0)
                for t in range(16):
                    pltpu.sync_copy(stage_sh.at[t, pl.ds(base, RPT)], tmp_v)

                    def acc(g, carry):
                        dsum_v[pl.ds(g * 16, 16)] = (
                            dsum_v[pl.ds(g * 16, 16)]
                            + tmp_v[pl.ds(g * 16, 16)])
                        return carry
                    lax.fori_loop(0, RPT // 16, acc, 0)
                pltpu.sync_copy(dsum_v, deg_out.at[pl.ds(base, RPT)])

        pltpu.sync_copy(agg_sh.at[pl.ds(base, RPT)],
                        agg_out.at[c, pl.ds(base, RPT)])

    return pl.kernel(body, out_type=out_type, mesh=mesh,
                     scratch_types=scratch)


def _dense_body(paired_out, agg_ref, deg_ref, xr_ref, wl_ref, wr_ref, b_ref,
                out_ref):
    aggf = jnp.concatenate([agg_ref[0], agg_ref[1]], axis=1)
    xf = jnp.concatenate([xr_ref[0], xr_ref[1]], axis=1)
    inv = 1.0 / jnp.maximum(deg_ref[...], 1.0)
    h = jnp.dot(aggf * inv, wl_ref[...], preferred_element_type=jnp.float32)
    h = h + b_ref[...] + jnp.dot(xf, wr_ref[...],
                                 preferred_element_type=jnp.float32)
    h = jnp.maximum(h, 0.0)
    if paired_out:
        out_ref[0] = h[:, :H]
        out_ref[1] = h[:, H:]
    else:
        out_ref[...] = h


def _make_dense(paired_out: bool):
    grid = (N_PAD // BN,)
    pair_spec = pl.BlockSpec((2, BN, H), lambda i: (0, i, 0))
    in_specs = [
        pair_spec,
        pl.BlockSpec((BN, 1), lambda i: (i, 0)),
        pair_spec,
        pl.BlockSpec((D, D), lambda i: (0, 0)),
        pl.BlockSpec((D, D), lambda i: (0, 0)),
        pl.BlockSpec((1, D), lambda i: (0, 0)),
    ]
    if paired_out:
        out_shape = jax.ShapeDtypeStruct((2, N_PAD, H), jnp.float32)
        out_specs = pair_spec
    else:
        out_shape = jax.ShapeDtypeStruct((N_PAD, D), jnp.float32)
        out_specs = pl.BlockSpec((BN, D), lambda i: (i, 0))
    return pl.pallas_call(
        functools.partial(_dense_body, paired_out),
        grid=grid, in_specs=in_specs, out_specs=out_specs,
        out_shape=out_shape)


_make_sc_aggregate = functools.lru_cache(None)(_make_sc_aggregate)
_make_dense = functools.lru_cache(None)(_make_dense)


def kernel(x, edge_index, W1_l, b1, W1_r, W2_l, b2, W2_r):
    src = edge_index[0].astype(jnp.int32)
    dst = edge_index[1].astype(jnp.int32)
    src_p = jnp.concatenate([src, jnp.zeros((E_PAD - E,), jnp.int32)])
    dst_p = jnp.concatenate([dst, jnp.full((E_PAD - E,), N, jnp.int32)])
    src2 = jnp.stack([src_p, src_p + N_PAD])
    src2 = src2.reshape(2, 16, GROUPS, GCHUNKS, CHUNK)
    dst3 = dst_p.reshape(16, GROUPS, GCHUNKS, CHUNK)

    xt = x.reshape(N, 2, H).transpose(1, 0, 2)
    xt = jnp.pad(xt, ((0, 0), (0, N_PAD - N), (0, 0)))
    x2n = xt.reshape(2 * N_PAD, H)

    agg1, deg = _make_sc_aggregate(True)(x2n, src2, dst3)
    deg_col = deg.reshape(N_PAD, 1)
    h2n = _make_dense(True)(agg1, deg_col, x2n.reshape(2, N_PAD, H),
                            W1_l.T, W1_r.T, b1.reshape(1, D))
    agg2, = _make_sc_aggregate(False)(h2n.reshape(2 * N_PAD, H), src2, dst3)
    out = _make_dense(False)(agg2, deg_col, h2n, W2_l.T, W2_r.T,
                             b2.reshape(1, D))
    return out[:N]


# speedup vs baseline: 3.6860x; 1.1501x over previous
import functools

import jax
import jax.numpy as jnp
from jax import lax
from jax.experimental import pallas as pl
from jax.experimental.pallas import tpu as pltpu
from jax.experimental.pallas import tpu_sc as plsc

N = 10000
E = 160000
D = 256
H = 128
N_PAD = 10240
RPT = N_PAD // 16
GROUPS = 4
E_PER_TILE = 10240
E_PAD = E_PER_TILE * 16
BN = 512


def _make_sc_aggregate(compute_deg: bool, chunk: int):
    gchunks = E_PER_TILE // (GROUPS * chunk)
    mesh = plsc.VectorSubcoreMesh(core_axis_name="c", subcore_axis_name="s",
                                  num_cores=2, num_subcores=16)
    out_type = [jax.ShapeDtypeStruct((2, N_PAD, H), jnp.float32)]
    scratch = [
        pltpu.VMEM((gchunks, chunk), jnp.int32),
        pltpu.VMEM((gchunks, chunk), jnp.int32),
        pltpu.VMEM((chunk, H), jnp.float32),
        pltpu.VMEM((chunk, H), jnp.float32),
        pltpu.VMEM_SHARED((N_PAD, H), jnp.float32),
        pltpu.SemaphoreType.DMA,
        pltpu.SemaphoreType.DMA,
    ]
    if compute_deg:
        out_type.append(jax.ShapeDtypeStruct((N_PAD,), jnp.float32))
        scratch += [
            pltpu.VMEM((N_PAD,), jnp.float32),
            pltpu.VMEM((RPT,), jnp.float32),
            pltpu.VMEM((RPT,), jnp.float32),
            pltpu.VMEM_SHARED((16, N_PAD), jnp.float32),
        ]

    def body(x2n, src_idx, dst_idx, *rest):
        if compute_deg:
            (agg_out, deg_out, src_v, dst_v, rows_a, rows_b, agg_sh,
             sem_a, sem_b, hist_v, tmp_v, dsum_v, stage_sh) = rest
        else:
            (agg_out, src_v, dst_v, rows_a, rows_b, agg_sh,
             sem_a, sem_b) = rest

        c = lax.axis_index("c")
        s = lax.axis_index("s")
        base = s * RPT
        zeros16 = jnp.zeros((16,), jnp.float32)
        for r in range(16):
            for k in range(H // 16):
                rows_a[r, pl.ds(k * 16, 16)] = zeros16

        if compute_deg:
            def zero_hist(g, carry):
                hist_v[pl.ds(g * 16, 16)] = zeros16
                return carry
            lax.fori_loop(0, N_PAD // 16, zero_hist, 0)

        def zero_agg(i, carry):
            pltpu.sync_copy(rows_a.at[pl.ds(0, 16)],
                            agg_sh.at[pl.ds(base + i * 16, 16)])
            return carry
        lax.fori_loop(0, RPT // 16, zero_agg, 0)

        plsc.subcore_barrier()

        def hist_chunk(j):
            if not compute_deg:
                return

            @pl.when(c == 0)
            def _():
                iota16 = lax.iota(jnp.int32, 16)
                one16 = jnp.ones((16,), jnp.float32)
                zero16 = jnp.zeros((16,), jnp.float32)
                for k in range(chunk // 16):
                    dvec = dst_v[j, pl.ds(k * 16, 16)]
                    for l in range(16):
                        idx = dvec[l]
                        wbase = lax.bitwise_and(idx, ~15)
                        lane = lax.bitwise_and(idx, 15)
                        oh = jnp.where(iota16 == lane, one16, zero16)
                        w = hist_v[pl.ds(wbase, 16)]
                        hist_v[pl.ds(wbase, 16)] = w + oh

        def wait(rows, sem):
            pltpu.make_async_copy(x2n.at[pl.ds(0, chunk)], rows, sem).wait()

        def group(g, carry):
            pltpu.sync_copy(src_idx.at[c, s, g], src_v)
            pltpu.sync_copy(dst_idx.at[s, g], dst_v)
            pltpu.async_copy(x2n.at[src_v.at[0]], rows_a, sem_a)

            def pair(t, carry2):
                j0 = t * 2
                pltpu.async_copy(x2n.at[src_v.at[j0 + 1]], rows_b, sem_b)
                hist_chunk(j0)
                wait(rows_a, sem_a)
                pltpu.sync_copy(rows_a, agg_sh.at[dst_v.at[j0]], add=True)

                @pl.when(j0 + 2 < gchunks)
                def _():
                    pltpu.async_copy(x2n.at[src_v.at[j0 + 2]], rows_a, sem_a)
                hist_chunk(j0 + 1)
                wait(rows_b, sem_b)
                pltpu.sync_copy(rows_b, agg_sh.at[dst_v.at[j0 + 1]], add=True)
                return carry2
            lax.fori_loop(0, gchunks // 2, pair, 0)
            return carry
        lax.fori_loop(0, GROUPS, group, 0)

        if compute_deg:
            @pl.when(c == 0)
            def _():
                pltpu.sync_copy(hist_v, stage_sh.at[s])
        plsc.subcore_barrier()

        if compute_deg:
            @pl.when(c == 0)
            def _():
                def zero_dsum(g, carry):
                    dsum_v[pl.ds(g * 16, 16)] = zeros16
                    return carry
                lax.fori_loop(0, RPT // 16, zero_dsum, 0)
                for t in range(16):
                    pltpu.sync_copy(stage_sh.at[t, pl.ds(base, RPT)], tmp_v)

                    def acc(g, carry):
                        dsum_v[pl.ds(g * 16, 16)] = (
                            dsum_v[pl.ds(g * 16, 16)]
                            + tmp_v[pl.ds(g * 16, 16)])
                        return carry
                    lax.fori_loop(0, RPT // 16, acc, 0)
                pltpu.sync_copy(dsum_v, deg_out.at[pl.ds(base, RPT)])

        pltpu.sync_copy(agg_sh.at[pl.ds(base, RPT)],
                        agg_out.at[c, pl.ds(base, RPT)])

    return pl.kernel(body, out_type=out_type, mesh=mesh,
                     scratch_types=scratch)


def _dense_body(paired_out, agg_ref, deg_ref, xr_ref, wl_ref, wr_ref, b_ref,
                out_ref):
    aggf = jnp.concatenate([agg_ref[0], agg_ref[1]], axis=1)
    xf = jnp.concatenate([xr_ref[0], xr_ref[1]], axis=1)
    inv = 1.0 / jnp.maximum(deg_ref[...], 1.0)
    h = jnp.dot(aggf * inv, wl_ref[...], preferred_element_type=jnp.float32)
    h = h + b_ref[...] + jnp.dot(xf, wr_ref[...],
                                 preferred_element_type=jnp.float32)
    h = jnp.maximum(h, 0.0)
    if paired_out:
        out_ref[0] = h[:, :H]
        out_ref[1] = h[:, H:]
    else:
        out_ref[...] = h


def _make_dense(paired_out: bool):
    grid = (N_PAD // BN,)
    pair_spec = pl.BlockSpec((2, BN, H), lambda i: (0, i, 0))
    in_specs = [
        pair_spec,
        pl.BlockSpec((BN, 1), lambda i: (i, 0)),
        pair_spec,
        pl.BlockSpec((D, D), lambda i: (0, 0)),
        pl.BlockSpec((D, D), lambda i: (0, 0)),
        pl.BlockSpec((1, D), lambda i: (0, 0)),
    ]
    if paired_out:
        out_shape = jax.ShapeDtypeStruct((2, N_PAD, H), jnp.float32)
        out_specs = pair_spec
    else:
        out_shape = jax.ShapeDtypeStruct((N_PAD, D), jnp.float32)
        out_specs = pl.BlockSpec((BN, D), lambda i: (i, 0))
    return pl.pallas_call(
        functools.partial(_dense_body, paired_out),
        grid=grid, in_specs=in_specs, out_specs=out_specs,
        out_shape=out_shape)


_make_sc_aggregate = functools.lru_cache(None)(_make_sc_aggregate)
_make_dense = functools.lru_cache(None)(_make_dense)

CHUNK1 = 64
CHUNK2 = 128


def kernel(x, edge_index, W1_l, b1, W1_r, W2_l, b2, W2_r):
    src = edge_index[0].astype(jnp.int32)
    dst = edge_index[1].astype(jnp.int32)
    src_p = jnp.concatenate([src, jnp.zeros((E_PAD - E,), jnp.int32)])
    dst_p = jnp.concatenate([dst, jnp.full((E_PAD - E,), N, jnp.int32)])
    src2 = jnp.stack([src_p, src_p + N_PAD])

    def idx4(a, chunk, lead):
        return a.reshape(lead + (16, GROUPS, E_PER_TILE // (GROUPS * chunk),
                                 chunk))

    xt = x.reshape(N, 2, H).transpose(1, 0, 2)
    xt = jnp.pad(xt, ((0, 0), (0, N_PAD - N), (0, 0)))
    x2n = xt.reshape(2 * N_PAD, H)

    agg1, deg = _make_sc_aggregate(True, CHUNK1)(
        x2n, idx4(src2, CHUNK1, (2,)), idx4(dst_p, CHUNK1, ()))
    deg_col = deg.reshape(N_PAD, 1)
    h2n = _make_dense(True)(agg1, deg_col, x2n.reshape(2, N_PAD, H),
                            W1_l.T, W1_r.T, b1.reshape(1, D))
    agg2, = _make_sc_aggregate(False, CHUNK2)(
        h2n.reshape(2 * N_PAD, H), idx4(src2, CHUNK2, (2,)),
        idx4(dst_p, CHUNK2, ()))
    out = _make_dense(False)(agg2, deg_col, h2n, W2_l.T, W2_r.T,
                             b2.reshape(1, D))
    return out[:N]
